# Initial kernel scaffold; baseline (speedup 1.0000x reference)
import jax
import jax.numpy as jnp
from jax.experimental import pallas as pl


def kernel(user_x, tag_x, edge_tt, edge_ut, edge_tu, params):
    raise NotImplementedError("write your pallas kernel here")


# speedup vs baseline: 3.2149x; 3.2149x over previous
import functools

import jax
import jax.numpy as jnp
from jax import lax
from jax.experimental import pallas as pl
from jax.experimental.pallas import tpu as pltpu
from jax.experimental.pallas import tpu_sc as plsc

H = 128
D = 256
NH = 4
EPS = 1e-5

NC = 2
NS = 16
NW = NC * NS
GU = 8


def _seg_body(n_pad, n_months, units_pm, refs):
    (table, srcg, dstl, zrow, out_s, acc, idx_s, idx_d, rows, sem) = refs

    c = lax.axis_index("c")
    s = lax.axis_index("s")
    w = s * NC + c

    upw = units_pm // NW
    rows_ps = n_pad // NS
    rb = pl.multiple_of(s * rows_ps, 8)

    def month(m, carry):
        pltpu.sync_copy(zrow.at[pl.ds(rb, rows_ps)], acc.at[pl.ds(rb, rows_ps)])
        plsc.subcore_barrier()

        ub = m * units_pm + w * upw

        def group(g, carry2):
            gb = pl.multiple_of(ub + g * GU, 8)
            pltpu.sync_copy(srcg.at[pl.ds(gb, GU)], idx_s)
            pltpu.sync_copy(dstl.at[pl.ds(gb, GU)], idx_d)

            def unit(j, carry3):
                pltpu.async_copy(table.at[idx_s.at[j]], rows, sem).wait()
                pltpu.sync_copy(rows, acc.at[idx_d.at[j]], add=True)
                return carry3

            lax.fori_loop(0, GU, unit, 0)
            return carry2

        lax.fori_loop(0, upw // GU, group, 0)

        plsc.subcore_barrier()
        pltpu.sync_copy(acc.at[pl.ds(rb, rows_ps)],
                        out_s.at[c, m, pl.ds(rb, rows_ps)])
        plsc.subcore_barrier()
        return carry

    lax.fori_loop(0, n_months, month, 0)


def _make_seg(n_pad, n_months, units_pm):
    out_type = [jax.ShapeDtypeStruct((NC, n_months, n_pad, H), jnp.float32)]
    scratch = [
        pltpu.VMEM_SHARED((n_pad, H), jnp.float32),
        pltpu.VMEM((GU, 128), jnp.int32),
        pltpu.VMEM((GU, 128), jnp.int32),
        pltpu.VMEM((128, H), jnp.float32),
        pltpu.SemaphoreType.DMA,
    ]

    mesh = plsc.VectorSubcoreMesh(core_axis_name="c", subcore_axis_name="s")

    def body(*refs):
        _seg_body(n_pad, n_months, units_pm, refs)

    return pl.kernel(body, out_type=out_type, mesh=mesh, scratch_types=scratch)


def _bn_proj(user_x, tag_x, p):
    M, N, _ = user_x.shape

    def body(ux, tx, ug, ub, tg, tb, uW, upb, tW, tpb, ou, ot):
        def bnp(x, g, b, W, pb):
            mu = jnp.mean(x, axis=0, keepdims=True)
            var = jnp.mean(x * x, axis=0, keepdims=True) - mu * mu
            xn = (x - mu) * lax.rsqrt(var + EPS) * g[...] + b[...]
            return jnp.dot(xn, W[...], preferred_element_type=jnp.float32) + pb[...]

        ou[0] = bnp(ux[0], ug, ub, uW, upb)
        ot[0] = bnp(tx[0], tg, tb, tW, tpb)

    full = lambda shape: pl.BlockSpec(shape, lambda m: (0,) * len(shape))
    xspec = pl.BlockSpec((1, N, H), lambda m: (m, 0, 0))
    vec = lambda: full((1, H))
    mat = lambda: full((H, H))
    return pl.pallas_call(
        body,
        grid=(M,),
        in_specs=[xspec, xspec, vec(), vec(), vec(), vec(), mat(), vec(), mat(), vec()],
        out_specs=[xspec, xspec],
        out_shape=[jax.ShapeDtypeStruct((M, N, H), jnp.float32)] * 2,
    )(
        user_x, tag_x,
        p["user_norm_g"].reshape(1, H), p["user_norm_b"].reshape(1, H),
        p["tag_norm_g"].reshape(1, H), p["tag_norm_b"].reshape(1, H),
        p["user_proj_W"], p["user_proj_b"].reshape(1, H),
        p["tag_proj_W"], p["tag_proj_b"].reshape(1, H),
    )


def _layer_combine(l, last, xt, xu, segs, cnts, p):
    M, N, _ = xt.shape
    NB = 10
    B = N // NB

    def body(xt_r, xu_r, stt0, stt1, sut0, sut1, stu0, stu1,
             ctt0, ctt1, cut0, cut1, ctu0, ctu1,
             Wl0, bl0, Wr0, Wl1, bl1, Wr1, Wl2, bl2, Wr2, *outs):
        xt_b = xt_r[0]
        xu_b = xu_r[0]

        def agg(s0, s1, c0, c1):
            cnt = c0[0][:, 0:1] + c1[0][:, 0:1]
            return (s0[0] + s1[0]) / jnp.maximum(cnt, 1.0)

        def sage(a, x, Wl, bl, Wr):
            return (jnp.dot(a, Wl[...], preferred_element_type=jnp.float32) + bl[...]
                    + jnp.dot(x, Wr[...], preferred_element_type=jnp.float32))

        tt = sage(agg(stt0, stt1, ctt0, ctt1), xt_b, Wl0, bl0, Wr0)
        ut = sage(agg(sut0, sut1, cut0, cut1), xt_b, Wl1, bl1, Wr1)
        tu = sage(agg(stu0, stu1, ctu0, ctu1), xu_b, Wl2, bl2, Wr2)
        xt_n = jnp.maximum((tt + ut) * 0.5, 0.0)
        xu_n = jnp.maximum(tu, 0.0)

        if last:
            (pool,) = outs
            b = pl.program_id(1)

            @pl.when(b == 0)
            def _():
                pool[...] = jnp.zeros_like(pool)

            su = jnp.sum(xu_n, axis=0, keepdims=True) * (1.0 / N)
            st = jnp.sum(xt_n, axis=0, keepdims=True) * (1.0 / N)
            pool[0] += jnp.concatenate([su, st], axis=1)
        else:
            oxt, oxu = outs
            oxt[0] = xt_n
            oxu[0] = xu_n

    xspec = pl.BlockSpec((1, B, H), lambda m, b: (m, b, 0))
    cspec = xspec
    mat = pl.BlockSpec((H, H), lambda m, b: (0, 0))
    vec = pl.BlockSpec((1, H), lambda m, b: (0, 0))

    stt, sut, stu = segs
    ctt, cut, ctu = cnts
    ins = [xt, xu,
           stt[0], stt[1], sut[0], sut[1], stu[0], stu[1],
           ctt[0], ctt[1], cut[0], cut[1], ctu[0], ctu[1]]
    in_specs = [xspec] * 8 + [cspec] * 6
    for r in range(3):
        ins += [p["sage_Wl"][l, r], p["sage_bl"][l, r].reshape(1, H), p["sage_Wr"][l, r]]
        in_specs += [mat, vec, mat]

    if last:
        out_specs = [pl.BlockSpec((1, 1, 2 * H), lambda m, b: (m, 0, 0))]
        out_shape = [jax.ShapeDtypeStruct((M, 1, 2 * H), jnp.float32)]
    else:
        out_specs = [xspec, xspec]
        out_shape = [jax.ShapeDtypeStruct((M, N, H), jnp.float32)] * 2

    return pl.pallas_call(
        body,
        grid=(M, NB),
        in_specs=in_specs,
        out_specs=out_specs,
        out_shape=out_shape,
    )(*ins)


def _transformer(pooled, p):
    M = pooled.shape[0]
    dh = D // NH

    ins = [pooled]
    for l in range(3):
        ins += [p["tr_Wq"][l], p["tr_bq"][l].reshape(1, D),
                p["tr_Wk"][l], p["tr_bk"][l].reshape(1, D),
                p["tr_Wv"][l], p["tr_bv"][l].reshape(1, D),
                p["tr_Wo"][l], p["tr_bo"][l].reshape(1, D),
                p["tr_ln1_g"][l].reshape(1, D), p["tr_ln1_b"][l].reshape(1, D),
                p["tr_W1"][l], p["tr_b1"][l].reshape(1, 256),
                p["tr_W2"][l], p["tr_b2"][l].reshape(1, D),
                p["tr_ln2_g"][l].reshape(1, D), p["tr_ln2_b"][l].reshape(1, D)]
    ins += [p["head_W"], p["head_b"].reshape(1, 4)]

    def body(x_r, *refs):
        wrefs = refs[:-1]
        o_r = refs[-1]
        x = x_r[...]

        def ln(v, g, b):
            mu = jnp.mean(v, axis=-1, keepdims=True)
            var = jnp.mean(v * v, axis=-1, keepdims=True) - mu * mu
            return (v - mu) * lax.rsqrt(var + EPS) * g[...] + b[...]

        k = 0
        for l in range(3):
            (Wq, bq, Wk, bk, Wv, bv, Wo, bo,
             g1, b1, W1, bf1, W2, bf2, g2, b2) = wrefs[k:k + 16]
            k += 16
            q = jnp.dot(x, Wq[...], preferred_element_type=jnp.float32) + bq[...]
            kk = jnp.dot(x, Wk[...], preferred_element_type=jnp.float32) + bk[...]
            v = jnp.dot(x, Wv[...], preferred_element_type=jnp.float32) + bv[...]
            heads = []
            for h in range(NH):
                qh = q[:, h * dh:(h + 1) * dh]
                kh = kk[:, h * dh:(h + 1) * dh]
                vh = v[:, h * dh:(h + 1) * dh]
                a = lax.dot_general(qh, kh, (((1,), (1,)), ((), ())),
                                    preferred_element_type=jnp.float32)
                a = jax.nn.softmax(a * (1.0 / jnp.sqrt(float(dh))), axis=-1)
                heads.append(jnp.dot(a, vh, preferred_element_type=jnp.float32))
            o = (jnp.dot(jnp.concatenate(heads, axis=1), Wo[...],
                         preferred_element_type=jnp.float32) + bo[...])
            x = ln(x + o, g1, b1)
            f = jnp.maximum(jnp.dot(x, W1[...], preferred_element_type=jnp.float32)
                            + bf1[...], 0.0)
            f = jnp.dot(f, W2[...], preferred_element_type=jnp.float32) + bf2[...]
            x = ln(x + f, g2, b2)

        hW, hb = wrefs[-2], wrefs[-1]
        r = x[M - 1:M, :]
        o_r[...] = lax.dot_general(r, hW[...], (((1,), (1,)), ((), ())),
                                   preferred_element_type=jnp.float32) + hb[...]

    body2 = lambda *rs: body(rs[0], *rs[1:])
    return pl.pallas_call(
        body2,
        out_shape=jax.ShapeDtypeStruct((1, 4), jnp.float32),
    )(*ins)


def kernel(user_x, tag_x, edge_tt, edge_ut, edge_tu, params):
    p = params
    M, N, _ = user_x.shape
    E = edge_tt.shape[2]
    assert E % 128 == 0
    units_pm = -(-(E // 128) // NW) * NW
    pad_e = units_pm * 128 - E
    n_pad = ((N + 127) // 128) * 128
    padr = n_pad - N

    offs = (jnp.arange(M, dtype=jnp.int32) * N)[:, None]
    fsrc = (jnp.arange(pad_e, dtype=jnp.int32) % N)[None, :] + offs
    fdst = jnp.broadcast_to(
        (N + jnp.arange(pad_e, dtype=jnp.int32) % padr)[None, :], (M, pad_e))

    def prep(e):
        src = jnp.concatenate([e[:, 0, :] + offs, fsrc], axis=1)
        dst = jnp.concatenate([e[:, 1, :], fdst], axis=1)
        return (src.reshape(M * units_pm, 128), dst.reshape(M * units_pm, 128))

    src_tt, dst_tt = prep(edge_tt)
    src_ut, dst_ut = prep(edge_ut)
    src_tu, dst_tu = prep(edge_tu)

    zrow = jnp.zeros((n_pad, H), jnp.float32)
    ones_tbl = jnp.ones((M * N, H), jnp.float32)

    seg = _make_seg(n_pad, M, units_pm)

    xu, xt = _bn_proj(user_x, tag_x, p)

    (ctt,) = seg(ones_tbl, src_tt, dst_tt, zrow)
    (cut,) = seg(ones_tbl, src_ut, dst_ut, zrow)
    (ctu,) = seg(ones_tbl, src_tu, dst_tu, zrow)
    cnts = (ctt, cut, ctu)

    pooled = None
    for l in range(3):
        tbl_t = xt.reshape(M * N, H)
        tbl_u = xu.reshape(M * N, H)
        (stt,) = seg(tbl_t, src_tt, dst_tt, zrow)
        (sut,) = seg(tbl_u, src_ut, dst_ut, zrow)
        (stu,) = seg(tbl_t, src_tu, dst_tu, zrow)
        if l < 2:
            xt, xu = _layer_combine(l, False, xt, xu, (stt, sut, stu), cnts, p)
        else:
            (pooled,) = _layer_combine(l, True, xt, xu, (stt, sut, stu), cnts, p)
            pooled = pooled.reshape(M, 2 * H)

    return _transformer(pooled, p)


# speedup vs baseline: 4.2220x; 1.3133x over previous
import functools

import jax
import jax.numpy as jnp
from jax import lax
from jax.experimental import pallas as pl
from jax.experimental.pallas import tpu as pltpu
from jax.experimental.pallas import tpu_sc as plsc

H = 128
D = 256
NH = 4
EPS = 1e-5

NC = 2
NS = 16
NW = NC * NS
GU = 8


def _seg_body(n_pad, n_months, units_pm, refs):
    (tbl_t, tbl_u,
     s_tt, d_tt, s_ut, d_ut, s_tu, d_tu,
     zrow, o_tt, o_ut, o_tu, acc, idx_s, idx_d, rows, sem) = refs
    outs = (o_tt, o_ut, o_tu)

    c = lax.axis_index("c")
    s = lax.axis_index("s")

    upw = units_pm // NS
    rows_ps = n_pad // NS
    rb = pl.multiple_of(s * rows_ps, 8)
    mh = n_months // NC

    for r, (table, srcg, dstl) in enumerate(
            [(tbl_t, s_tt, d_tt), (tbl_u, s_ut, d_ut), (tbl_t, s_tu, d_tu)]):

        def month(i, carry, srcg=srcg, dstl=dstl, table=table, r=r):
            m = c * mh + i
            pltpu.sync_copy(zrow.at[pl.ds(rb, rows_ps)],
                            acc.at[pl.ds(rb, rows_ps)])
            plsc.subcore_barrier()

            ub = m * units_pm + s * upw

            def group(g, carry2):
                gb = pl.multiple_of(ub + g * GU, 8)
                pltpu.sync_copy(srcg.at[pl.ds(gb, GU)], idx_s)
                pltpu.sync_copy(dstl.at[pl.ds(gb, GU)], idx_d)

                def unit(j, carry3):
                    pltpu.async_copy(table.at[idx_s.at[j]], rows, sem).wait()
                    pltpu.sync_copy(rows, acc.at[idx_d.at[j]], add=True)
                    return carry3

                lax.fori_loop(0, GU, unit, 0)
                return carry2

            lax.fori_loop(0, upw // GU, group, 0)

            plsc.subcore_barrier()
            pltpu.sync_copy(acc.at[pl.ds(rb, rows_ps)],
                            outs[r].at[m, pl.ds(rb, rows_ps)])
            plsc.subcore_barrier()
            return carry

        lax.fori_loop(0, mh, month, 0)


def _make_seg(n_pad, n_months, units_pm):
    out_type = [jax.ShapeDtypeStruct((n_months, n_pad, H), jnp.float32)] * 3
    scratch = [
        pltpu.VMEM_SHARED((n_pad, H), jnp.float32),
        pltpu.VMEM((GU, 128), jnp.int32),
        pltpu.VMEM((GU, 128), jnp.int32),
        pltpu.VMEM((128, H), jnp.float32),
        pltpu.SemaphoreType.DMA,
    ]

    mesh = plsc.VectorSubcoreMesh(core_axis_name="c", subcore_axis_name="s")

    def body(*refs):
        _seg_body(n_pad, n_months, units_pm, refs)

    return pl.kernel(body, out_type=out_type, mesh=mesh, scratch_types=scratch)


def _cnt_body(n_pad, n_months, units_pm, refs):
    (ones128, d_tt, d_ut, d_tu, zrow, o_tt, o_ut, o_tu,
     acc, idx_d, rows, sem) = refs
    outs = (o_tt, o_ut, o_tu)

    c = lax.axis_index("c")
    s = lax.axis_index("s")

    upw = units_pm // NS
    rows_ps = n_pad // NS
    rb = pl.multiple_of(s * rows_ps, 8)
    mh = n_months // NC

    pltpu.sync_copy(ones128, rows)

    for r, dstl in enumerate([d_tt, d_ut, d_tu]):

        def month(i, carry, dstl=dstl, r=r):
            m = c * mh + i
            pltpu.sync_copy(zrow.at[pl.ds(rb, rows_ps)],
                            acc.at[pl.ds(rb, rows_ps)])
            plsc.subcore_barrier()

            ub = m * units_pm + s * upw

            def group(g, carry2):
                gb = pl.multiple_of(ub + g * GU, 8)
                pltpu.sync_copy(dstl.at[pl.ds(gb, GU)], idx_d)

                def unit(j, carry3):
                    pltpu.sync_copy(rows, acc.at[idx_d.at[j]], add=True)
                    return carry3

                lax.fori_loop(0, GU, unit, 0)
                return carry2

            lax.fori_loop(0, upw // GU, group, 0)

            plsc.subcore_barrier()
            pltpu.sync_copy(acc.at[pl.ds(rb, rows_ps)],
                            outs[r].at[m, pl.ds(rb, rows_ps)])
            plsc.subcore_barrier()
            return carry

        lax.fori_loop(0, mh, month, 0)


def _make_cnt(n_pad, n_months, units_pm):
    out_type = [jax.ShapeDtypeStruct((n_months, n_pad, H), jnp.float32)] * 3
    scratch = [
        pltpu.VMEM_SHARED((n_pad, H), jnp.float32),
        pltpu.VMEM((GU, 128), jnp.int32),
        pltpu.VMEM((128, H), jnp.float32),
        pltpu.SemaphoreType.DMA,
    ]

    mesh = plsc.VectorSubcoreMesh(core_axis_name="c", subcore_axis_name="s")

    def body(*refs):
        _cnt_body(n_pad, n_months, units_pm, refs)

    return pl.kernel(body, out_type=out_type, mesh=mesh, scratch_types=scratch)


def _bn_proj(user_x, tag_x, p):
    M, N, _ = user_x.shape

    def body(ux, tx, ug, ub, tg, tb, uW, upb, tW, tpb, ou, ot):
        def bnp(x, g, b, W, pb):
            mu = jnp.mean(x, axis=0, keepdims=True)
            var = jnp.mean(x * x, axis=0, keepdims=True) - mu * mu
            xn = (x - mu) * lax.rsqrt(var + EPS) * g[...] + b[...]
            return jnp.dot(xn, W[...], preferred_element_type=jnp.float32) + pb[...]

        ou[0] = bnp(ux[0], ug, ub, uW, upb)
        ot[0] = bnp(tx[0], tg, tb, tW, tpb)

    full = lambda shape: pl.BlockSpec(shape, lambda m: (0,) * len(shape))
    xspec = pl.BlockSpec((1, N, H), lambda m: (m, 0, 0))
    vec = lambda: full((1, H))
    mat = lambda: full((H, H))
    return pl.pallas_call(
        body,
        grid=(M,),
        in_specs=[xspec, xspec, vec(), vec(), vec(), vec(), mat(), vec(), mat(), vec()],
        out_specs=[xspec, xspec],
        out_shape=[jax.ShapeDtypeStruct((M, N, H), jnp.float32)] * 2,
    )(
        user_x, tag_x,
        p["user_norm_g"].reshape(1, H), p["user_norm_b"].reshape(1, H),
        p["tag_norm_g"].reshape(1, H), p["tag_norm_b"].reshape(1, H),
        p["user_proj_W"], p["user_proj_b"].reshape(1, H),
        p["tag_proj_W"], p["tag_proj_b"].reshape(1, H),
    )


def _layer_combine(l, last, xt, xu, segs, cnts, p):
    M, N, _ = xt.shape
    NB = 10
    B = N // NB

    def body(xt_r, xu_r, stt, sut, stu, ctt, cut, ctu,
             Wl0, bl0, Wr0, Wl1, bl1, Wr1, Wl2, bl2, Wr2, *outs):
        xt_b = xt_r[0]
        xu_b = xu_r[0]

        def agg(s0, c0):
            cnt = c0[0][:, 0:1]
            return s0[0] / jnp.maximum(cnt, 1.0)

        def sage(a, x, Wl, bl, Wr):
            return (jnp.dot(a, Wl[...], preferred_element_type=jnp.float32) + bl[...]
                    + jnp.dot(x, Wr[...], preferred_element_type=jnp.float32))

        tt = sage(agg(stt, ctt), xt_b, Wl0, bl0, Wr0)
        ut = sage(agg(sut, cut), xt_b, Wl1, bl1, Wr1)
        tu = sage(agg(stu, ctu), xu_b, Wl2, bl2, Wr2)
        xt_n = jnp.maximum((tt + ut) * 0.5, 0.0)
        xu_n = jnp.maximum(tu, 0.0)

        if last:
            (pool,) = outs
            b = pl.program_id(1)

            @pl.when(b == 0)
            def _():
                pool[...] = jnp.zeros_like(pool)

            su = jnp.sum(xu_n, axis=0, keepdims=True) * (1.0 / N)
            st = jnp.sum(xt_n, axis=0, keepdims=True) * (1.0 / N)
            pool[0] += jnp.concatenate([su, st], axis=1)
        else:
            oxt, oxu = outs
            oxt[0] = xt_n
            oxu[0] = xu_n

    xspec = pl.BlockSpec((1, B, H), lambda m, b: (m, b, 0))
    mat = pl.BlockSpec((H, H), lambda m, b: (0, 0))
    vec = pl.BlockSpec((1, H), lambda m, b: (0, 0))

    ins = [xt, xu, *segs, *cnts]
    in_specs = [xspec] * 8
    for r in range(3):
        ins += [p["sage_Wl"][l, r], p["sage_bl"][l, r].reshape(1, H), p["sage_Wr"][l, r]]
        in_specs += [mat, vec, mat]

    if last:
        out_specs = [pl.BlockSpec((1, 1, 2 * H), lambda m, b: (m, 0, 0))]
        out_shape = [jax.ShapeDtypeStruct((M, 1, 2 * H), jnp.float32)]
    else:
        out_specs = [xspec, xspec]
        out_shape = [jax.ShapeDtypeStruct((M, N, H), jnp.float32)] * 2

    return pl.pallas_call(
        body,
        grid=(M, NB),
        in_specs=in_specs,
        out_specs=out_specs,
        out_shape=out_shape,
    )(*ins)


def _transformer(pooled, p):
    M = pooled.shape[0]
    dh = D // NH

    ins = [pooled]
    for l in range(3):
        ins += [p["tr_Wq"][l], p["tr_bq"][l].reshape(1, D),
                p["tr_Wk"][l], p["tr_bk"][l].reshape(1, D),
                p["tr_Wv"][l], p["tr_bv"][l].reshape(1, D),
                p["tr_Wo"][l], p["tr_bo"][l].reshape(1, D),
                p["tr_ln1_g"][l].reshape(1, D), p["tr_ln1_b"][l].reshape(1, D),
                p["tr_W1"][l], p["tr_b1"][l].reshape(1, 256),
                p["tr_W2"][l], p["tr_b2"][l].reshape(1, D),
                p["tr_ln2_g"][l].reshape(1, D), p["tr_ln2_b"][l].reshape(1, D)]
    ins += [p["head_W"], p["head_b"].reshape(1, 4)]

    def body(x_r, *refs):
        wrefs = refs[:-1]
        o_r = refs[-1]
        x = x_r[...]

        def ln(v, g, b):
            mu = jnp.mean(v, axis=-1, keepdims=True)
            var = jnp.mean(v * v, axis=-1, keepdims=True) - mu * mu
            return (v - mu) * lax.rsqrt(var + EPS) * g[...] + b[...]

        k = 0
        for l in range(3):
            (Wq, bq, Wk, bk, Wv, bv, Wo, bo,
             g1, b1, W1, bf1, W2, bf2, g2, b2) = wrefs[k:k + 16]
            k += 16
            q = jnp.dot(x, Wq[...], preferred_element_type=jnp.float32) + bq[...]
            kk = jnp.dot(x, Wk[...], preferred_element_type=jnp.float32) + bk[...]
            v = jnp.dot(x, Wv[...], preferred_element_type=jnp.float32) + bv[...]
            heads = []
            for h in range(NH):
                qh = q[:, h * dh:(h + 1) * dh]
                kh = kk[:, h * dh:(h + 1) * dh]
                vh = v[:, h * dh:(h + 1) * dh]
                a = lax.dot_general(qh, kh, (((1,), (1,)), ((), ())),
                                    preferred_element_type=jnp.float32)
                a = jax.nn.softmax(a * (1.0 / jnp.sqrt(float(dh))), axis=-1)
                heads.append(jnp.dot(a, vh, preferred_element_type=jnp.float32))
            o = (jnp.dot(jnp.concatenate(heads, axis=1), Wo[...],
                         preferred_element_type=jnp.float32) + bo[...])
            x = ln(x + o, g1, b1)
            f = jnp.maximum(jnp.dot(x, W1[...], preferred_element_type=jnp.float32)
                            + bf1[...], 0.0)
            f = jnp.dot(f, W2[...], preferred_element_type=jnp.float32) + bf2[...]
            x = ln(x + f, g2, b2)

        hW, hb = wrefs[-2], wrefs[-1]
        r = x[M - 1:M, :]
        o_r[...] = lax.dot_general(r, hW[...], (((1,), (1,)), ((), ())),
                                   preferred_element_type=jnp.float32) + hb[...]

    body2 = lambda *rs: body(rs[0], *rs[1:])
    return pl.pallas_call(
        body2,
        out_shape=jax.ShapeDtypeStruct((1, 4), jnp.float32),
    )(*ins)


def kernel(user_x, tag_x, edge_tt, edge_ut, edge_tu, params):
    p = params
    M, N, _ = user_x.shape
    E = edge_tt.shape[2]
    assert E % 128 == 0 and M % NC == 0
    gpn = NS * GU
    units_pm = -(-(E // 128) // gpn) * gpn
    pad_e = units_pm * 128 - E
    n_pad = ((N + 127) // 128) * 128
    padr = n_pad - N

    offs = (jnp.arange(M, dtype=jnp.int32) * N)[:, None]
    fsrc = (jnp.arange(pad_e, dtype=jnp.int32) % N)[None, :] + offs
    fdst = jnp.broadcast_to(
        (N + jnp.arange(pad_e, dtype=jnp.int32) % padr)[None, :], (M, pad_e))

    def prep(e):
        src = jnp.concatenate([e[:, 0, :] + offs, fsrc], axis=1)
        dst = jnp.concatenate([e[:, 1, :], fdst], axis=1)
        return (src.reshape(M * units_pm, 128), dst.reshape(M * units_pm, 128))

    src_tt, dst_tt = prep(edge_tt)
    src_ut, dst_ut = prep(edge_ut)
    src_tu, dst_tu = prep(edge_tu)

    zrow = jnp.zeros((n_pad, H), jnp.float32)
    ones128 = jnp.ones((128, H), jnp.float32)

    seg = _make_seg(n_pad, M, units_pm)
    cntk = _make_cnt(n_pad, M, units_pm)

    xu, xt = _bn_proj(user_x, tag_x, p)

    cnts = cntk(ones128, dst_tt, dst_ut, dst_tu, zrow)

    pooled = None
    for l in range(3):
        tbl_t = xt.reshape(M * N, H)
        tbl_u = xu.reshape(M * N, H)
        segs = seg(tbl_t, tbl_u, src_tt, dst_tt, src_ut, dst_ut,
                   src_tu, dst_tu, zrow)
        if l < 2:
            xt, xu = _layer_combine(l, False, xt, xu, segs, cnts, p)
        else:
            (pooled,) = _layer_combine(l, True, xt, xu, segs, cnts, p)
            pooled = pooled.reshape(M, 2 * H)

    return _transformer(pooled, p)


# speedup vs baseline: 5.5847x; 1.3228x over previous
import functools

import jax
import jax.numpy as jnp
from jax import lax
from jax.experimental import pallas as pl
from jax.experimental.pallas import tpu as pltpu
from jax.experimental.pallas import tpu_sc as plsc

H = 128
D = 256
NH = 4
EPS = 1e-5

NC = 2
NS = 16
NW = NC * NS
GU = 8


def _seg_body(n_pad, n_months, units_pm, refs):
    (tbl_t, tbl_u,
     s_tt, d_tt, s_ut, d_ut, s_tu, d_tu,
     zrow, o_tt, o_ut, o_tu, acc, idx_s, idx_d,
     rows0, rows1, sem0, sem1) = refs
    outs = (o_tt, o_ut, o_tu)
    rbufs = (rows0, rows1)
    sems = (sem0, sem1)

    c = lax.axis_index("c")
    s = lax.axis_index("s")

    upw = units_pm // NS
    rows_ps = n_pad // NS
    rb = pl.multiple_of(s * rows_ps, 8)
    mh = n_months // NC

    for r, (table, srcg, dstl) in enumerate(
            [(tbl_t, s_tt, d_tt), (tbl_u, s_ut, d_ut), (tbl_t, s_tu, d_tu)]):

        def month(i, carry, srcg=srcg, dstl=dstl, table=table, r=r):
            m = c * mh + i
            pltpu.sync_copy(zrow.at[pl.ds(rb, rows_ps)],
                            acc.at[pl.ds(rb, rows_ps)])
            plsc.subcore_barrier()

            ub = m * units_pm + s * upw

            def group(g, carry2):
                gb = pl.multiple_of(ub + g * GU, 8)
                pltpu.sync_copy(srcg.at[pl.ds(gb, GU)], idx_s)
                pltpu.sync_copy(dstl.at[pl.ds(gb, GU)], idx_d)

                cur = pltpu.async_copy(table.at[idx_s.at[0]], rbufs[0], sems[0])
                for j in range(GU):
                    nxt = None
                    if j + 1 < GU:
                        nxt = pltpu.async_copy(table.at[idx_s.at[j + 1]],
                                               rbufs[(j + 1) % 2],
                                               sems[(j + 1) % 2])
                    cur.wait()
                    pltpu.sync_copy(rbufs[j % 2], acc.at[idx_d.at[j]], add=True)
                    cur = nxt
                return carry2

            lax.fori_loop(0, upw // GU, group, 0)

            plsc.subcore_barrier()
            pltpu.sync_copy(acc.at[pl.ds(rb, rows_ps)],
                            outs[r].at[m, pl.ds(rb, rows_ps)])
            plsc.subcore_barrier()
            return carry

        lax.fori_loop(0, mh, month, 0)


def _make_seg(n_pad, n_months, units_pm):
    out_type = [jax.ShapeDtypeStruct((n_months, n_pad, H), jnp.float32)] * 3
    scratch = [
        pltpu.VMEM_SHARED((n_pad, H), jnp.float32),
        pltpu.VMEM((GU, 128), jnp.int32),
        pltpu.VMEM((GU, 128), jnp.int32),
        pltpu.VMEM((128, H), jnp.float32),
        pltpu.VMEM((128, H), jnp.float32),
        pltpu.SemaphoreType.DMA,
        pltpu.SemaphoreType.DMA,
    ]

    mesh = plsc.VectorSubcoreMesh(core_axis_name="c", subcore_axis_name="s")

    def body(*refs):
        _seg_body(n_pad, n_months, units_pm, refs)

    return pl.kernel(body, out_type=out_type, mesh=mesh, scratch_types=scratch)


def _cnt_body(n_pad, n_months, units_pm, refs):
    (ones128, d_tt, d_ut, d_tu, zrow, o_tt, o_ut, o_tu,
     acc, idx_d, rows, sem) = refs
    outs = (o_tt, o_ut, o_tu)

    c = lax.axis_index("c")
    s = lax.axis_index("s")

    upw = units_pm // NS
    rows_ps = n_pad // NS
    rb = pl.multiple_of(s * rows_ps, 8)
    mh = n_months // NC

    pltpu.sync_copy(ones128, rows)

    for r, dstl in enumerate([d_tt, d_ut, d_tu]):

        def month(i, carry, dstl=dstl, r=r):
            m = c * mh + i
            pltpu.sync_copy(zrow.at[pl.ds(rb, rows_ps)],
                            acc.at[pl.ds(rb, rows_ps)])
            plsc.subcore_barrier()

            ub = m * units_pm + s * upw

            def group(g, carry2):
                gb = pl.multiple_of(ub + g * GU, 8)
                pltpu.sync_copy(dstl.at[pl.ds(gb, GU)], idx_d)

                def unit(j, carry3):
                    pltpu.sync_copy(rows, acc.at[idx_d.at[j]], add=True)
                    return carry3

                lax.fori_loop(0, GU, unit, 0)
                return carry2

            lax.fori_loop(0, upw // GU, group, 0)

            plsc.subcore_barrier()
            pltpu.sync_copy(acc.at[pl.ds(rb, rows_ps)],
                            outs[r].at[m, pl.ds(rb, rows_ps)])
            plsc.subcore_barrier()
            return carry

        lax.fori_loop(0, mh, month, 0)


def _make_cnt(n_pad, n_months, units_pm):
    out_type = [jax.ShapeDtypeStruct((n_months, n_pad, H), jnp.float32)] * 3
    scratch = [
        pltpu.VMEM_SHARED((n_pad, H), jnp.float32),
        pltpu.VMEM((GU, 128), jnp.int32),
        pltpu.VMEM((128, H), jnp.float32),
        pltpu.SemaphoreType.DMA,
    ]

    mesh = plsc.VectorSubcoreMesh(core_axis_name="c", subcore_axis_name="s")

    def body(*refs):
        _cnt_body(n_pad, n_months, units_pm, refs)

    return pl.kernel(body, out_type=out_type, mesh=mesh, scratch_types=scratch)


def _bn_proj(user_x, tag_x, p):
    M, N, _ = user_x.shape

    def body(ux, tx, ug, ub, tg, tb, uW, upb, tW, tpb, ou, ot):
        def bnp(x, g, b, W, pb):
            mu = jnp.mean(x, axis=0, keepdims=True)
            var = jnp.mean(x * x, axis=0, keepdims=True) - mu * mu
            xn = (x - mu) * lax.rsqrt(var + EPS) * g[...] + b[...]
            return jnp.dot(xn, W[...], preferred_element_type=jnp.float32) + pb[...]

        ou[0] = bnp(ux[0], ug, ub, uW, upb)
        ot[0] = bnp(tx[0], tg, tb, tW, tpb)

    full = lambda shape: pl.BlockSpec(shape, lambda m: (0,) * len(shape))
    xspec = pl.BlockSpec((1, N, H), lambda m: (m, 0, 0))
    vec = lambda: full((1, H))
    mat = lambda: full((H, H))
    return pl.pallas_call(
        body,
        grid=(M,),
        in_specs=[xspec, xspec, vec(), vec(), vec(), vec(), mat(), vec(), mat(), vec()],
        out_specs=[xspec, xspec],
        out_shape=[jax.ShapeDtypeStruct((M, N, H), jnp.float32)] * 2,
    )(
        user_x, tag_x,
        p["user_norm_g"].reshape(1, H), p["user_norm_b"].reshape(1, H),
        p["tag_norm_g"].reshape(1, H), p["tag_norm_b"].reshape(1, H),
        p["user_proj_W"], p["user_proj_b"].reshape(1, H),
        p["tag_proj_W"], p["tag_proj_b"].reshape(1, H),
    )


def _layer_combine(l, last, xt, xu, segs, cnts, p):
    M, N, _ = xt.shape
    NB = 10
    B = N // NB

    def body(xt_r, xu_r, stt, sut, stu, ctt, cut, ctu,
             Wl0, bl0, Wr0, Wl1, bl1, Wr1, Wl2, bl2, Wr2, *outs):
        xt_b = xt_r[0]
        xu_b = xu_r[0]

        def agg(s0, c0):
            cnt = c0[0][:, 0:1]
            return s0[0] / jnp.maximum(cnt, 1.0)

        def sage(a, x, Wl, bl, Wr):
            return (jnp.dot(a, Wl[...], preferred_element_type=jnp.float32) + bl[...]
                    + jnp.dot(x, Wr[...], preferred_element_type=jnp.float32))

        tt = sage(agg(stt, ctt), xt_b, Wl0, bl0, Wr0)
        ut = sage(agg(sut, cut), xt_b, Wl1, bl1, Wr1)
        tu = sage(agg(stu, ctu), xu_b, Wl2, bl2, Wr2)
        xt_n = jnp.maximum((tt + ut) * 0.5, 0.0)
        xu_n = jnp.maximum(tu, 0.0)

        if last:
            (pool,) = outs
            b = pl.program_id(1)

            @pl.when(b == 0)
            def _():
                pool[...] = jnp.zeros_like(pool)

            su = jnp.sum(xu_n, axis=0, keepdims=True) * (1.0 / N)
            st = jnp.sum(xt_n, axis=0, keepdims=True) * (1.0 / N)
            pool[0] += jnp.concatenate([su, st], axis=1)
        else:
            oxt, oxu = outs
            oxt[0] = xt_n
            oxu[0] = xu_n

    xspec = pl.BlockSpec((1, B, H), lambda m, b: (m, b, 0))
    mat = pl.BlockSpec((H, H), lambda m, b: (0, 0))
    vec = pl.BlockSpec((1, H), lambda m, b: (0, 0))

    ins = [xt, xu, *segs, *cnts]
    in_specs = [xspec] * 8
    for r in range(3):
        ins += [p["sage_Wl"][l, r], p["sage_bl"][l, r].reshape(1, H), p["sage_Wr"][l, r]]
        in_specs += [mat, vec, mat]

    if last:
        out_specs = [pl.BlockSpec((1, 1, 2 * H), lambda m, b: (m, 0, 0))]
        out_shape = [jax.ShapeDtypeStruct((M, 1, 2 * H), jnp.float32)]
    else:
        out_specs = [xspec, xspec]
        out_shape = [jax.ShapeDtypeStruct((M, N, H), jnp.float32)] * 2

    return pl.pallas_call(
        body,
        grid=(M, NB),
        in_specs=in_specs,
        out_specs=out_specs,
        out_shape=out_shape,
    )(*ins)


def _transformer(pooled, p):
    M = pooled.shape[0]
    dh = D // NH

    ins = [pooled]
    for l in range(3):
        ins += [p["tr_Wq"][l], p["tr_bq"][l].reshape(1, D),
                p["tr_Wk"][l], p["tr_bk"][l].reshape(1, D),
                p["tr_Wv"][l], p["tr_bv"][l].reshape(1, D),
                p["tr_Wo"][l], p["tr_bo"][l].reshape(1, D),
                p["tr_ln1_g"][l].reshape(1, D), p["tr_ln1_b"][l].reshape(1, D),
                p["tr_W1"][l], p["tr_b1"][l].reshape(1, 256),
                p["tr_W2"][l], p["tr_b2"][l].reshape(1, D),
                p["tr_ln2_g"][l].reshape(1, D), p["tr_ln2_b"][l].reshape(1, D)]
    ins += [p["head_W"], p["head_b"].reshape(1, 4)]

    def body(x_r, *refs):
        wrefs = refs[:-1]
        o_r = refs[-1]
        x = x_r[...]

        def ln(v, g, b):
            mu = jnp.mean(v, axis=-1, keepdims=True)
            var = jnp.mean(v * v, axis=-1, keepdims=True) - mu * mu
            return (v - mu) * lax.rsqrt(var + EPS) * g[...] + b[...]

        k = 0
        for l in range(3):
            (Wq, bq, Wk, bk, Wv, bv, Wo, bo,
             g1, b1, W1, bf1, W2, bf2, g2, b2) = wrefs[k:k + 16]
            k += 16
            q = jnp.dot(x, Wq[...], preferred_element_type=jnp.float32) + bq[...]
            kk = jnp.dot(x, Wk[...], preferred_element_type=jnp.float32) + bk[...]
            v = jnp.dot(x, Wv[...], preferred_element_type=jnp.float32) + bv[...]
            heads = []
            for h in range(NH):
                qh = q[:, h * dh:(h + 1) * dh]
                kh = kk[:, h * dh:(h + 1) * dh]
                vh = v[:, h * dh:(h + 1) * dh]
                a = lax.dot_general(qh, kh, (((1,), (1,)), ((), ())),
                                    preferred_element_type=jnp.float32)
                a = jax.nn.softmax(a * (1.0 / jnp.sqrt(float(dh))), axis=-1)
                heads.append(jnp.dot(a, vh, preferred_element_type=jnp.float32))
            o = (jnp.dot(jnp.concatenate(heads, axis=1), Wo[...],
                         preferred_element_type=jnp.float32) + bo[...])
            x = ln(x + o, g1, b1)
            f = jnp.maximum(jnp.dot(x, W1[...], preferred_element_type=jnp.float32)
                            + bf1[...], 0.0)
            f = jnp.dot(f, W2[...], preferred_element_type=jnp.float32) + bf2[...]
            x = ln(x + f, g2, b2)

        hW, hb = wrefs[-2], wrefs[-1]
        r = x[M - 1:M, :]
        o_r[...] = lax.dot_general(r, hW[...], (((1,), (1,)), ((), ())),
                                   preferred_element_type=jnp.float32) + hb[...]

    body2 = lambda *rs: body(rs[0], *rs[1:])
    return pl.pallas_call(
        body2,
        out_shape=jax.ShapeDtypeStruct((1, 4), jnp.float32),
    )(*ins)


def kernel(user_x, tag_x, edge_tt, edge_ut, edge_tu, params):
    p = params
    M, N, _ = user_x.shape
    E = edge_tt.shape[2]
    assert E % 128 == 0 and M % NC == 0
    gpn = NS * GU
    units_pm = -(-(E // 128) // gpn) * gpn
    pad_e = units_pm * 128 - E
    n_pad = ((N + 127) // 128) * 128
    padr = n_pad - N

    offs = (jnp.arange(M, dtype=jnp.int32) * N)[:, None]
    fsrc = (jnp.arange(pad_e, dtype=jnp.int32) % N)[None, :] + offs
    fdst = jnp.broadcast_to(
        (N + jnp.arange(pad_e, dtype=jnp.int32) % padr)[None, :], (M, pad_e))

    def prep(e):
        src = jnp.concatenate([e[:, 0, :] + offs, fsrc], axis=1)
        dst = jnp.concatenate([e[:, 1, :], fdst], axis=1)
        return (src.reshape(M * units_pm, 128), dst.reshape(M * units_pm, 128))

    src_tt, dst_tt = prep(edge_tt)
    src_ut, dst_ut = prep(edge_ut)
    src_tu, dst_tu = prep(edge_tu)

    zrow = jnp.zeros((n_pad, H), jnp.float32)
    ones128 = jnp.ones((128, H), jnp.float32)

    seg = _make_seg(n_pad, M, units_pm)
    cntk = _make_cnt(n_pad, M, units_pm)

    xu, xt = _bn_proj(user_x, tag_x, p)

    cnts = cntk(ones128, dst_tt, dst_ut, dst_tu, zrow)

    pooled = None
    for l in range(3):
        tbl_t = xt.reshape(M * N, H)
        tbl_u = xu.reshape(M * N, H)
        segs = seg(tbl_t, tbl_u, src_tt, dst_tt, src_ut, dst_ut,
                   src_tu, dst_tu, zrow)
        if l < 2:
            xt, xu = _layer_combine(l, False, xt, xu, segs, cnts, p)
        else:
            (pooled,) = _layer_combine(l, True, xt, xu, segs, cnts, p)
            pooled = pooled.reshape(M, 2 * H)

    return _transformer(pooled, p)


# speedup vs baseline: 6.0305x; 1.0798x over previous
import functools

import jax
import jax.numpy as jnp
from jax import lax
from jax.experimental import pallas as pl
from jax.experimental.pallas import tpu as pltpu
from jax.experimental.pallas import tpu_sc as plsc

H = 128
D = 256
NH = 4
EPS = 1e-5

NC = 2
NS = 16
NW = NC * NS
GU = 16


def _seg_body(n_pad, n_months, units_pm, refs):
    (tbl_t, tbl_u,
     s_tt, d_tt, s_ut, d_ut, s_tu, d_tu,
     zrow, o_tt, o_ut, o_tu, acc, idx_s, idx_d,
     rows0, rows1, sem0, sem1) = refs
    outs = (o_tt, o_ut, o_tu)
    rbufs = (rows0, rows1)
    sems = (sem0, sem1)

    c = lax.axis_index("c")
    s = lax.axis_index("s")

    upw = units_pm // NS
    rows_ps = n_pad // NS
    rb = pl.multiple_of(s * rows_ps, 8)
    mh = n_months // NC

    for r, (table, srcg, dstl) in enumerate(
            [(tbl_t, s_tt, d_tt), (tbl_u, s_ut, d_ut), (tbl_t, s_tu, d_tu)]):

        def month(i, carry, srcg=srcg, dstl=dstl, table=table, r=r):
            m = c * mh + i
            pltpu.sync_copy(zrow.at[pl.ds(rb, rows_ps)],
                            acc.at[pl.ds(rb, rows_ps)])
            plsc.subcore_barrier()

            ub = m * units_pm + s * upw

            def group(g, carry2):
                gb = pl.multiple_of(ub + g * GU, 8)
                pltpu.sync_copy(srcg.at[pl.ds(gb, GU)], idx_s)
                pltpu.sync_copy(dstl.at[pl.ds(gb, GU)], idx_d)

                cur = pltpu.async_copy(table.at[idx_s.at[0]], rbufs[0], sems[0])
                for j in range(GU):
                    nxt = None
                    if j + 1 < GU:
                        nxt = pltpu.async_copy(table.at[idx_s.at[j + 1]],
                                               rbufs[(j + 1) % 2],
                                               sems[(j + 1) % 2])
                    cur.wait()
                    pltpu.sync_copy(rbufs[j % 2], acc.at[idx_d.at[j]], add=True)
                    cur = nxt
                return carry2

            lax.fori_loop(0, upw // GU, group, 0)

            plsc.subcore_barrier()
            pltpu.sync_copy(acc.at[pl.ds(rb, rows_ps)],
                            outs[r].at[m, pl.ds(rb, rows_ps)])
            plsc.subcore_barrier()
            return carry

        lax.fori_loop(0, mh, month, 0)


def _make_seg(n_pad, n_months, units_pm):
    out_type = [jax.ShapeDtypeStruct((n_months, n_pad, H), jnp.float32)] * 3
    scratch = [
        pltpu.VMEM_SHARED((n_pad, H), jnp.float32),
        pltpu.VMEM((GU, 128), jnp.int32),
        pltpu.VMEM((GU, 128), jnp.int32),
        pltpu.VMEM((128, H), jnp.float32),
        pltpu.VMEM((128, H), jnp.float32),
        pltpu.SemaphoreType.DMA,
        pltpu.SemaphoreType.DMA,
    ]

    mesh = plsc.VectorSubcoreMesh(core_axis_name="c", subcore_axis_name="s")

    def body(*refs):
        _seg_body(n_pad, n_months, units_pm, refs)

    return pl.kernel(body, out_type=out_type, mesh=mesh, scratch_types=scratch)


def _cnt_body(n_pad, n_months, units_pm, refs):
    (ones128, d_tt, d_ut, d_tu, zrow, o_tt, o_ut, o_tu,
     acc, idx_d, rows, sem) = refs
    outs = (o_tt, o_ut, o_tu)

    c = lax.axis_index("c")
    s = lax.axis_index("s")

    upw = units_pm // NS
    rows_ps = n_pad // NS
    rb = pl.multiple_of(s * rows_ps, 8)
    mh = n_months // NC

    pltpu.sync_copy(ones128, rows)

    for r, dstl in enumerate([d_tt, d_ut, d_tu]):

        def month(i, carry, dstl=dstl, r=r):
            m = c * mh + i
            pltpu.sync_copy(zrow.at[pl.ds(rb, rows_ps)],
                            acc.at[pl.ds(rb, rows_ps)])
            plsc.subcore_barrier()

            ub = m * units_pm + s * upw

            def group(g, carry2):
                gb = pl.multiple_of(ub + g * GU, 8)
                pltpu.sync_copy(dstl.at[pl.ds(gb, GU)], idx_d)

                def unit(j, carry3):
                    pltpu.sync_copy(rows, acc.at[idx_d.at[j]], add=True)
                    return carry3

                lax.fori_loop(0, GU, unit, 0)
                return carry2

            lax.fori_loop(0, upw // GU, group, 0)

            plsc.subcore_barrier()
            pltpu.sync_copy(acc.at[pl.ds(rb, rows_ps)],
                            outs[r].at[m, pl.ds(rb, rows_ps)])
            plsc.subcore_barrier()
            return carry

        lax.fori_loop(0, mh, month, 0)


def _make_cnt(n_pad, n_months, units_pm):
    out_type = [jax.ShapeDtypeStruct((n_months, n_pad, H), jnp.float32)] * 3
    scratch = [
        pltpu.VMEM_SHARED((n_pad, H), jnp.float32),
        pltpu.VMEM((GU, 128), jnp.int32),
        pltpu.VMEM((128, H), jnp.float32),
        pltpu.SemaphoreType.DMA,
    ]

    mesh = plsc.VectorSubcoreMesh(core_axis_name="c", subcore_axis_name="s")

    def body(*refs):
        _cnt_body(n_pad, n_months, units_pm, refs)

    return pl.kernel(body, out_type=out_type, mesh=mesh, scratch_types=scratch)


def _bn_proj(user_x, tag_x, p):
    M, N, _ = user_x.shape

    def body(ux, tx, ug, ub, tg, tb, uW, upb, tW, tpb, ou, ot):
        def bnp(x, g, b, W, pb):
            mu = jnp.mean(x, axis=0, keepdims=True)
            var = jnp.mean(x * x, axis=0, keepdims=True) - mu * mu
            xn = (x - mu) * lax.rsqrt(var + EPS) * g[...] + b[...]
            return jnp.dot(xn, W[...], preferred_element_type=jnp.float32) + pb[...]

        ou[0] = bnp(ux[0], ug, ub, uW, upb)
        ot[0] = bnp(tx[0], tg, tb, tW, tpb)

    full = lambda shape: pl.BlockSpec(shape, lambda m: (0,) * len(shape))
    xspec = pl.BlockSpec((1, N, H), lambda m: (m, 0, 0))
    vec = lambda: full((1, H))
    mat = lambda: full((H, H))
    return pl.pallas_call(
        body,
        grid=(M,),
        in_specs=[xspec, xspec, vec(), vec(), vec(), vec(), mat(), vec(), mat(), vec()],
        out_specs=[xspec, xspec],
        out_shape=[jax.ShapeDtypeStruct((M, N, H), jnp.float32)] * 2,
    )(
        user_x, tag_x,
        p["user_norm_g"].reshape(1, H), p["user_norm_b"].reshape(1, H),
        p["tag_norm_g"].reshape(1, H), p["tag_norm_b"].reshape(1, H),
        p["user_proj_W"], p["user_proj_b"].reshape(1, H),
        p["tag_proj_W"], p["tag_proj_b"].reshape(1, H),
    )


def _layer_combine(l, last, xt, xu, segs, cnts, p):
    M, N, _ = xt.shape
    NB = 10
    B = N // NB

    def body(xt_r, xu_r, stt, sut, stu, ctt, cut, ctu,
             Wl0, bl0, Wr0, Wl1, bl1, Wr1, Wl2, bl2, Wr2, *outs):
        xt_b = xt_r[0]
        xu_b = xu_r[0]

        def agg(s0, c0):
            cnt = c0[0][:, 0:1]
            return s0[0] / jnp.maximum(cnt, 1.0)

        def sage(a, x, Wl, bl, Wr):
            return (jnp.dot(a, Wl[...], preferred_element_type=jnp.float32) + bl[...]
                    + jnp.dot(x, Wr[...], preferred_element_type=jnp.float32))

        tt = sage(agg(stt, ctt), xt_b, Wl0, bl0, Wr0)
        ut = sage(agg(sut, cut), xt_b, Wl1, bl1, Wr1)
        tu = sage(agg(stu, ctu), xu_b, Wl2, bl2, Wr2)
        xt_n = jnp.maximum((tt + ut) * 0.5, 0.0)
        xu_n = jnp.maximum(tu, 0.0)

        if last:
            (pool,) = outs
            b = pl.program_id(1)

            @pl.when(b == 0)
            def _():
                pool[...] = jnp.zeros_like(pool)

            su = jnp.sum(xu_n, axis=0, keepdims=True) * (1.0 / N)
            st = jnp.sum(xt_n, axis=0, keepdims=True) * (1.0 / N)
            pool[0] += jnp.concatenate([su, st], axis=1)
        else:
            oxt, oxu = outs
            oxt[0] = xt_n
            oxu[0] = xu_n

    xspec = pl.BlockSpec((1, B, H), lambda m, b: (m, b, 0))
    mat = pl.BlockSpec((H, H), lambda m, b: (0, 0))
    vec = pl.BlockSpec((1, H), lambda m, b: (0, 0))

    ins = [xt, xu, *segs, *cnts]
    in_specs = [xspec] * 8
    for r in range(3):
        ins += [p["sage_Wl"][l, r], p["sage_bl"][l, r].reshape(1, H), p["sage_Wr"][l, r]]
        in_specs += [mat, vec, mat]

    if last:
        out_specs = [pl.BlockSpec((1, 1, 2 * H), lambda m, b: (m, 0, 0))]
        out_shape = [jax.ShapeDtypeStruct((M, 1, 2 * H), jnp.float32)]
    else:
        out_specs = [xspec, xspec]
        out_shape = [jax.ShapeDtypeStruct((M, N, H), jnp.float32)] * 2

    return pl.pallas_call(
        body,
        grid=(M, NB),
        in_specs=in_specs,
        out_specs=out_specs,
        out_shape=out_shape,
    )(*ins)


def _transformer(pooled, p):
    M = pooled.shape[0]
    dh = D // NH

    ins = [pooled]
    for l in range(3):
        ins += [p["tr_Wq"][l], p["tr_bq"][l].reshape(1, D),
                p["tr_Wk"][l], p["tr_bk"][l].reshape(1, D),
                p["tr_Wv"][l], p["tr_bv"][l].reshape(1, D),
                p["tr_Wo"][l], p["tr_bo"][l].reshape(1, D),
                p["tr_ln1_g"][l].reshape(1, D), p["tr_ln1_b"][l].reshape(1, D),
                p["tr_W1"][l], p["tr_b1"][l].reshape(1, 256),
                p["tr_W2"][l], p["tr_b2"][l].reshape(1, D),
                p["tr_ln2_g"][l].reshape(1, D), p["tr_ln2_b"][l].reshape(1, D)]
    ins += [p["head_W"], p["head_b"].reshape(1, 4)]

    def body(x_r, *refs):
        wrefs = refs[:-1]
        o_r = refs[-1]
        x = x_r[...]

        def ln(v, g, b):
            mu = jnp.mean(v, axis=-1, keepdims=True)
            var = jnp.mean(v * v, axis=-1, keepdims=True) - mu * mu
            return (v - mu) * lax.rsqrt(var + EPS) * g[...] + b[...]

        k = 0
        for l in range(3):
            (Wq, bq, Wk, bk, Wv, bv, Wo, bo,
             g1, b1, W1, bf1, W2, bf2, g2, b2) = wrefs[k:k + 16]
            k += 16
            q = jnp.dot(x, Wq[...], preferred_element_type=jnp.float32) + bq[...]
            kk = jnp.dot(x, Wk[...], preferred_element_type=jnp.float32) + bk[...]
            v = jnp.dot(x, Wv[...], preferred_element_type=jnp.float32) + bv[...]
            heads = []
            for h in range(NH):
                qh = q[:, h * dh:(h + 1) * dh]
                kh = kk[:, h * dh:(h + 1) * dh]
                vh = v[:, h * dh:(h + 1) * dh]
                a = lax.dot_general(qh, kh, (((1,), (1,)), ((), ())),
                                    preferred_element_type=jnp.float32)
                a = jax.nn.softmax(a * (1.0 / jnp.sqrt(float(dh))), axis=-1)
                heads.append(jnp.dot(a, vh, preferred_element_type=jnp.float32))
            o = (jnp.dot(jnp.concatenate(heads, axis=1), Wo[...],
                         preferred_element_type=jnp.float32) + bo[...])
            x = ln(x + o, g1, b1)
            f = jnp.maximum(jnp.dot(x, W1[...], preferred_element_type=jnp.float32)
                            + bf1[...], 0.0)
            f = jnp.dot(f, W2[...], preferred_element_type=jnp.float32) + bf2[...]
            x = ln(x + f, g2, b2)

        hW, hb = wrefs[-2], wrefs[-1]
        r = x[M - 1:M, :]
        o_r[...] = lax.dot_general(r, hW[...], (((1,), (1,)), ((), ())),
                                   preferred_element_type=jnp.float32) + hb[...]

    body2 = lambda *rs: body(rs[0], *rs[1:])
    return pl.pallas_call(
        body2,
        out_shape=jax.ShapeDtypeStruct((1, 4), jnp.float32),
    )(*ins)


def kernel(user_x, tag_x, edge_tt, edge_ut, edge_tu, params):
    p = params
    M, N, _ = user_x.shape
    E = edge_tt.shape[2]
    assert E % 128 == 0 and M % NC == 0
    gpn = NS * GU
    units_pm = -(-(E // 128) // gpn) * gpn
    pad_e = units_pm * 128 - E
    n_pad = ((N + 127) // 128) * 128
    padr = n_pad - N

    offs = (jnp.arange(M, dtype=jnp.int32) * N)[:, None]
    fsrc = (jnp.arange(pad_e, dtype=jnp.int32) % N)[None, :] + offs
    fdst = jnp.broadcast_to(
        (N + jnp.arange(pad_e, dtype=jnp.int32) % padr)[None, :], (M, pad_e))

    def prep(e):
        src = jnp.concatenate([e[:, 0, :] + offs, fsrc], axis=1)
        dst = jnp.concatenate([e[:, 1, :], fdst], axis=1)
        return (src.reshape(M * units_pm, 128), dst.reshape(M * units_pm, 128))

    src_tt, dst_tt = prep(edge_tt)
    src_ut, dst_ut = prep(edge_ut)
    src_tu, dst_tu = prep(edge_tu)

    zrow = jnp.zeros((n_pad, H), jnp.float32)
    ones128 = jnp.ones((128, H), jnp.float32)

    seg = _make_seg(n_pad, M, units_pm)
    cntk = _make_cnt(n_pad, M, units_pm)

    xu, xt = _bn_proj(user_x, tag_x, p)

    cnts = cntk(ones128, dst_tt, dst_ut, dst_tu, zrow)

    pooled = None
    for l in range(3):
        tbl_t = xt.reshape(M * N, H)
        tbl_u = xu.reshape(M * N, H)
        segs = seg(tbl_t, tbl_u, src_tt, dst_tt, src_ut, dst_ut,
                   src_tu, dst_tu, zrow)
        if l < 2:
            xt, xu = _layer_combine(l, False, xt, xu, segs, cnts, p)
        else:
            (pooled,) = _layer_combine(l, True, xt, xu, segs, cnts, p)
            pooled = pooled.reshape(M, 2 * H)

    return _transformer(pooled, p)
